# baseline (speedup 1.0000x reference)
import functools

import jax
import jax.numpy as jnp
from jax import lax
from jax.experimental import pallas as pl
from jax.experimental.pallas import tpu as pltpu
from jax.experimental.pallas import tpu_sc as plsc

B, T, V, D = 4, 200, 8192, 1024
PPW = 8
ACTIVE = T // PPW
RPW = B * PPW
CHUNKS = 2
PPC = PPW // CHUNKS
RPC = B * PPC
L = 16

_mesh = plsc.VectorSubcoreMesh(core_axis_name="c", subcore_axis_name="s")


@functools.partial(
    pl.kernel,
    mesh=_mesh,
    out_type=jax.ShapeDtypeStruct((B, T * D), jnp.float32),
    scratch_types=[
        pltpu.VMEM((B, T), jnp.int32),
        pltpu.VMEM((RPW,), jnp.int32),
        pltpu.VMEM((RPW, D), jnp.float32),
        pltpu.SemaphoreType.DMA,
        pltpu.SemaphoreType.DMA,
        pltpu.SemaphoreType.DMA,
    ],
)
def _sc_gather(table_hbm, tok_hbm, out_hbm, tok_v, idx_v, rows_v, gs0, gs1, wsem):
    wid = lax.axis_index("s") * 2 + lax.axis_index("c")

    @pl.when(wid < ACTIVE)
    def _():
        t0 = wid * PPW
        pltpu.sync_copy(tok_hbm, tok_v)

        t0a = (wid // 2) * L
        r = (wid % 2) * PPW
        vb = [tok_v[b, pl.ds(t0a, L)] for b in range(B)]
        lane = lax.iota(jnp.int32, L)
        bsel = lax.rem(lane, B)
        gsems = [gs0, gs1]

        def _gather(c):
            return pltpu.make_async_copy(
                table_hbm.at[idx_v.at[pl.ds(c * RPC, RPC)]],
                rows_v.at[pl.ds(c * RPC, RPC)],
                gsems[c],
            )

        def _write(j):
            return pltpu.make_async_copy(
                rows_v.at[pl.ds(j * B, B)],
                out_hbm.at[:, pl.ds((t0 + j) * D, D)],
                wsem,
            )

        dnums = lax.GatherDimensionNumbers(
            offset_dims=(), collapsed_slice_dims=(0,), start_index_map=(0,)
        )
        for c in range(CHUNKS):
            pos = r + c * PPC + lax.div(lane, B)
            spread = [
                lax.gather(
                    v,
                    pos[:, None],
                    dnums,
                    (1,),
                    mode=lax.GatherScatterMode.PROMISE_IN_BOUNDS,
                )
                for v in vb
            ]
            mix = spread[B - 1]
            for b in range(B - 2, -1, -1):
                mix = jnp.where(bsel == b, spread[b], mix)
            idx_v[pl.ds(c * L, L)] = mix + bsel * V
            _gather(c).start()

        for c in range(CHUNKS):
            _gather(c).wait()
            for jj in range(PPC):
                _write(c * PPC + jj).start()
        pltpu.make_async_copy(
            rows_v, table_hbm.at[pl.ds(0, RPW)], wsem
        ).wait()


def kernel(sequence_embedding, tokens):
    table = sequence_embedding.reshape(B * V, D)
    return _sc_gather(table, tokens)


# speedup vs baseline: 1.0173x; 1.0173x over previous
import functools

import jax
import jax.numpy as jnp
from jax import lax
from jax.experimental import pallas as pl
from jax.experimental.pallas import tpu as pltpu
from jax.experimental.pallas import tpu_sc as plsc

B, T, V, D = 4, 200, 8192, 1024
PPW = 8
ACTIVE = T // PPW
RPW = B * PPW
CHUNKS = 2
PPC = PPW // CHUNKS
RPC = B * PPC
L = 16

_mesh = plsc.VectorSubcoreMesh(core_axis_name="c", subcore_axis_name="s")


@functools.partial(
    pl.kernel,
    mesh=_mesh,
    out_type=jax.ShapeDtypeStruct((B, T * D), jnp.float32),
    scratch_types=[
        pltpu.VMEM((B, T), jnp.int32),
        pltpu.VMEM((RPW,), jnp.int32),
        pltpu.VMEM((RPW, D), jnp.float32),
        pltpu.SemaphoreType.DMA,
        pltpu.SemaphoreType.DMA,
        pltpu.SemaphoreType.DMA,
    ],
)
def _sc_gather(table_hbm, tok_hbm, out_hbm, tok_v, idx_v, rows_v, gs0, gs1, wsem):
    wid = lax.axis_index("s") * 2 + lax.axis_index("c")

    @pl.when(wid < ACTIVE)
    def _():
        t0 = wid * PPW
        pltpu.sync_copy(tok_hbm, tok_v)

        t0a = (wid // 2) * L
        r = (wid % 2) * PPW
        vb = [tok_v[b, pl.ds(t0a, L)] for b in range(B)]
        lane = lax.iota(jnp.int32, L)
        bsel = lax.rem(lane, B)
        gsems = [gs0, gs1]

        def _gather(c):
            return pltpu.make_async_copy(
                table_hbm.at[idx_v.at[pl.ds(c * RPC, RPC)]],
                rows_v.at[pl.ds(c * RPC, RPC)],
                gsems[c],
            )

        def _write(j):
            return pltpu.make_async_copy(
                rows_v.at[pl.ds(j * B, B)],
                out_hbm.at[:, pl.ds((t0 + j) * D, D)],
                wsem,
            )

        dnums = lax.GatherDimensionNumbers(
            offset_dims=(), collapsed_slice_dims=(0,), start_index_map=(0,)
        )
        for c in range(CHUNKS):
            pos = r + c * PPC + lax.div(lane, B)
            spread = [
                lax.gather(
                    v,
                    pos[:, None],
                    dnums,
                    (1,),
                    mode=lax.GatherScatterMode.PROMISE_IN_BOUNDS,
                )
                for v in vb
            ]
            mix = spread[B - 1]
            for b in range(B - 2, -1, -1):
                mix = jnp.where(bsel == b, spread[b], mix)
            idx_v[pl.ds(c * L, L)] = mix + bsel * V

        for c in range(CHUNKS):
            _gather(c).start()
        for c in range(CHUNKS):
            _gather(c).wait()
            for jj in range(PPC):
                _write(c * PPC + jj).start()
        for j in range(PPW):
            _write(j).wait()


def kernel(sequence_embedding, tokens):
    table = sequence_embedding.reshape(B * V, D)
    return _sc_gather(table, tokens)
